# Initial kernel scaffold; baseline (speedup 1.0000x reference)
import jax
import jax.numpy as jnp
from jax.experimental import pallas as pl


def kernel(nt, tr, es, ed, ef, ne_w, te_w, ef_w, w_ih, w_hh, b_ih, b_hh, ng, nb, W1, b1, g2, bt2, W2, b2):
    raise NotImplementedError("write your pallas kernel here")


# speedup vs baseline: 25.7220x; 25.7220x over previous
import jax
import jax.numpy as jnp
from jax import lax
from jax.experimental import pallas as pl

_DIM = 128
_N = 32
_NE = 2048
_F32 = jnp.float32


def _tc_body(es_ref, ed_ref, ef_ref, nt_ref, tr_ref,
             ne_w_ref, te_w_ref, ef_w_ref,
             w_ih_ref, w_hh_ref, b_ih_ref, b_hh_ref, ng_ref, nb_ref,
             W1_ref, b1_ref, g2_ref, bt2_ref, W2_ref, b2_ref, out_ref):
    es = es_ref[:]
    ed = ed_ref[:]
    ef = ef_ref[:]
    mask = (ed < _N) & (es < _N)
    es_safe = jnp.where(mask, es, 0)
    ed_safe = jnp.where(mask, ed, 0)
    maskf = mask.astype(_F32)

    iota_n = lax.broadcasted_iota(jnp.int32, (_N, _NE), 0)
    iota_k = lax.broadcasted_iota(jnp.int32, (6, _NE), 0)
    oh_dst = (ed_safe == iota_n).astype(_F32) * maskf
    oh_src = (es_safe == iota_n).astype(_F32)
    oh_ef = (ef == iota_k).astype(_F32)

    nt_dims = (((1,), (1,)), ((), ()))
    A = lax.dot_general(oh_dst, oh_src, nt_dims,
                        preferred_element_type=_F32)
    F = lax.dot_general(oh_dst, oh_ef, nt_dims,
                        preferred_element_type=_F32)
    E = jnp.dot(F, ef_w_ref[:], preferred_element_type=_F32)
    cnt = jnp.sum(A, axis=1, keepdims=True)
    inv_cnt = 1.0 / jnp.maximum(cnt, 1.0)

    nt_c = nt_ref[:]
    tr_c = tr_ref[:]
    oh_nt = (nt_c == lax.broadcasted_iota(jnp.int32, (_N, 20), 1)).astype(_F32)
    oh_tr = (tr_c == lax.broadcasted_iota(jnp.int32, (_N, 6), 1)).astype(_F32)
    h = (jnp.dot(oh_nt, ne_w_ref[:], preferred_element_type=_F32)
         + jnp.dot(oh_tr, te_w_ref[:], preferred_element_type=_F32))

    w_ih = w_ih_ref[:]
    w_hh = w_hh_ref[:]
    b_ih = b_ih_ref[:]
    b_hh = b_hh_ref[:]
    ng = ng_ref[:]
    nb = nb_ref[:]

    for _ in range(5):
        agg = (jnp.dot(A, h, preferred_element_type=_F32) + E) * inv_cnt
        gi = lax.dot_general(agg, w_ih, nt_dims,
                             preferred_element_type=_F32) + b_ih
        gh = lax.dot_general(h, w_hh, nt_dims,
                             preferred_element_type=_F32) + b_hh
        r = jax.nn.sigmoid(gi[:, 0:128] + gh[:, 0:128])
        z = jax.nn.sigmoid(gi[:, 128:256] + gh[:, 128:256])
        n = jnp.tanh(gi[:, 256:384] + r * gh[:, 256:384])
        hn = (1.0 - z) * n + z * h
        mu = jnp.mean(hn, axis=1, keepdims=True)
        var = jnp.mean((hn - mu) ** 2, axis=1, keepdims=True)
        h = (hn - mu) / jnp.sqrt(var + 1e-5) * ng + nb

    hmean = jnp.mean(h, axis=0, keepdims=True)
    hmax = jnp.max(h, axis=0, keepdims=True)
    pooled = jnp.concatenate([hmean, hmax], axis=1)
    x = lax.dot_general(pooled, W1_ref[:], nt_dims,
                        preferred_element_type=_F32) + b1_ref[:]
    mu = jnp.mean(x, axis=1, keepdims=True)
    var = jnp.mean((x - mu) ** 2, axis=1, keepdims=True)
    x = (x - mu) / jnp.sqrt(var + 1e-5) * g2_ref[:] + bt2_ref[:]
    x = jnp.maximum(x, 0.0)
    out_row = x * W2_ref[:] + b2_ref[:] * (1.0 / 128.0)
    out_ref[:, :] = jnp.sum(out_row, axis=1, keepdims=True)


def kernel(nt, tr, es, ed, ef, ne_w, te_w, ef_w, w_ih, w_hh, b_ih, b_hh,
           ng, nb, W1, b1, g2, bt2, W2, b2):
    out = pl.pallas_call(
        _tc_body,
        out_shape=jax.ShapeDtypeStruct((1, 1), _F32),
    )(
        es.reshape(1, _NE), ed.reshape(1, _NE), ef.reshape(1, _NE),
        nt.reshape(_N, 1), tr.reshape(_N, 1),
        ne_w, te_w, ef_w,
        w_ih, w_hh, b_ih.reshape(1, 3 * _DIM), b_hh.reshape(1, 3 * _DIM),
        ng.reshape(1, _DIM), nb.reshape(1, _DIM),
        W1, b1.reshape(1, _DIM), g2.reshape(1, _DIM), bt2.reshape(1, _DIM),
        W2, jnp.broadcast_to(b2.reshape(1, 1), (1, _DIM)),
    )
    return out.reshape(())
